# baseline (device time: 129261 ns/iter reference)
import functools

import jax
import jax.numpy as jnp
from jax import lax
from jax.experimental import pallas as pl
from jax.experimental.pallas import tpu as pltpu

N_DEV = 8
N_TOK = 2048
D_IN = 512
D_OUT = 1024
E_LOCAL = 4
CHUNK = N_TOK // N_DEV


def kernel(x, router_W, route_idx, expert_W):
    def body(
        x_ref,
        router_ref,
        route_ref,
        ew_ref,
        out_ref,
        partial_ref,
        stage_ref,
        rs_send,
        rs_recv,
        ag_send,
        ag_recv,
    ):
        my = lax.axis_index("i")
        left = lax.rem(my + N_DEV - 1, N_DEV)
        right = lax.rem(my + 1, N_DEV)

        bsem = pltpu.get_barrier_semaphore()
        for nbr in (left, right):
            pl.semaphore_signal(
                bsem, inc=1, device_id=(nbr,), device_id_type=pl.DeviceIdType.MESH
            )
        pl.semaphore_wait(bsem, 2)

        xb = x_ref[...].astype(jnp.bfloat16)
        r = route_ref[...]
        acc = jnp.zeros((N_TOK, D_OUT), jnp.float32)
        for e in range(E_LOCAL):
            ge = my * E_LOCAL + e
            mask = r == ge
            xm = jnp.where(mask, xb, jnp.zeros_like(xb))
            wb = ew_ref[e, :, :].astype(jnp.bfloat16)
            acc = acc + jnp.dot(xm, wb, preferred_element_type=jnp.float32)
        accb = acc.astype(jnp.bfloat16)
        for c in range(N_DEV):
            partial_ref[c, :, :] = accb[c * CHUNK : (c + 1) * CHUNK, :]

        for h in range(N_DEV - 1):
            cs = lax.rem(my + N_DEV - h, N_DEV)
            cr = lax.rem(my + 2 * N_DEV - h - 1, N_DEV)
            rdma = pltpu.make_async_remote_copy(
                src_ref=partial_ref.at[cs],
                dst_ref=stage_ref.at[h],
                send_sem=rs_send.at[h],
                recv_sem=rs_recv.at[h],
                device_id=(right,),
                device_id_type=pl.DeviceIdType.MESH,
            )
            rdma.start()
            rdma.wait()
            partial_ref[cr, :, :] = partial_ref[cr, :, :] + stage_ref[h, :, :]

        cp = lax.rem(my + 1, N_DEV)
        out_ref[pl.ds(cp * CHUNK, CHUNK), :] = partial_ref[cp, :, :].astype(
            jnp.float32
        )

        for g in range(N_DEV - 1):
            cs = lax.rem(my + 1 + N_DEV - g, N_DEV)
            cr = lax.rem(my + N_DEV - g, N_DEV)
            rdma = pltpu.make_async_remote_copy(
                src_ref=partial_ref.at[cs],
                dst_ref=partial_ref.at[cs],
                send_sem=ag_send.at[g],
                recv_sem=ag_recv.at[g],
                device_id=(right,),
                device_id_type=pl.DeviceIdType.MESH,
            )
            rdma.start()
            rdma.wait()
            out_ref[pl.ds(cr * CHUNK, CHUNK), :] = partial_ref[cr, :, :].astype(
                jnp.float32
            )

        @functools.partial(
            pl.run_scoped, second_barrier=pltpu.SemaphoreType.REGULAR
        )
        def _(second_barrier):
            for nbr in (left, right):
                pl.semaphore_signal(
                    second_barrier,
                    inc=1,
                    device_id=(nbr,),
                    device_id_type=pl.DeviceIdType.MESH,
                )
            pl.semaphore_wait(second_barrier, 2)

    return pl.pallas_call(
        body,
        out_shape=jax.ShapeDtypeStruct((N_TOK, D_OUT), jnp.float32),
        in_specs=[pl.BlockSpec(memory_space=pltpu.VMEM)] * 4,
        out_specs=pl.BlockSpec(memory_space=pltpu.VMEM),
        scratch_shapes=[
            pltpu.VMEM((N_DEV, CHUNK, D_OUT), jnp.bfloat16),
            pltpu.VMEM((N_DEV - 1, CHUNK, D_OUT), jnp.bfloat16),
            pltpu.SemaphoreType.DMA((N_DEV - 1,)),
            pltpu.SemaphoreType.DMA((N_DEV - 1,)),
            pltpu.SemaphoreType.DMA((N_DEV - 1,)),
            pltpu.SemaphoreType.DMA((N_DEV - 1,)),
        ],
        compiler_params=pltpu.CompilerParams(collective_id=0),
    )(x, router_W, route_idx, expert_W)


# device time: 95795 ns/iter; 1.3494x vs baseline; 1.3494x over previous
import functools

import jax
import jax.numpy as jnp
from jax import lax
from jax.experimental import pallas as pl
from jax.experimental.pallas import tpu as pltpu

N_DEV = 8
N_TOK = 2048
D_IN = 512
D_OUT = 1024
E_LOCAL = 4
N_SUB = 2 * N_DEV
SUB = N_TOK // N_SUB


def kernel(x, router_W, route_idx, expert_W):
    def body(
        x_ref,
        router_ref,
        route_ref,
        ew_ref,
        out_ref,
        partial_ref,
        stage_r_ref,
        stage_l_ref,
        rs_r_send,
        rs_r_recv,
        rs_l_send,
        rs_l_recv,
        ag_r_send,
        ag_r_recv,
        ag_l_send,
        ag_l_recv,
    ):
        my = lax.axis_index("i")
        left = lax.rem(my + N_DEV - 1, N_DEV)
        right = lax.rem(my + 1, N_DEV)

        bsem = pltpu.get_barrier_semaphore()
        for nbr in (left, right):
            pl.semaphore_signal(
                bsem, inc=1, device_id=(nbr,), device_id_type=pl.DeviceIdType.MESH
            )
        pl.semaphore_wait(bsem, 2)

        xb = x_ref[...].astype(jnp.bfloat16)
        r = route_ref[...]
        acc = jnp.zeros((N_TOK, D_OUT), jnp.float32)
        for e in range(E_LOCAL):
            ge = my * E_LOCAL + e
            mask = r == ge
            xm = jnp.where(mask, xb, jnp.zeros_like(xb))
            wb = ew_ref[e, :, :].astype(jnp.bfloat16)
            acc = acc + jnp.dot(xm, wb, preferred_element_type=jnp.float32)
        accb = acc.astype(jnp.bfloat16)
        for s in range(N_SUB):
            partial_ref[s, :, :] = accb[s * SUB : (s + 1) * SUB, :]

        pending = []

        for h in range(N_DEV - 1):
            cs_r = 2 * lax.rem(my + N_DEV - h, N_DEV)
            cr_r = 2 * lax.rem(my + 2 * N_DEV - h - 1, N_DEV)
            cs_l = 2 * lax.rem(my + h, N_DEV) + 1
            cr_l = 2 * lax.rem(my + h + 1, N_DEV) + 1
            rd_r = pltpu.make_async_remote_copy(
                src_ref=partial_ref.at[cs_r],
                dst_ref=stage_r_ref.at[h],
                send_sem=rs_r_send.at[h],
                recv_sem=rs_r_recv.at[h],
                device_id=(right,),
                device_id_type=pl.DeviceIdType.MESH,
            )
            rd_l = pltpu.make_async_remote_copy(
                src_ref=partial_ref.at[cs_l],
                dst_ref=stage_l_ref.at[h],
                send_sem=rs_l_send.at[h],
                recv_sem=rs_l_recv.at[h],
                device_id=(left,),
                device_id_type=pl.DeviceIdType.MESH,
            )
            rd_r.start()
            rd_l.start()
            rd_r.wait_recv()
            partial_ref[cr_r, :, :] = partial_ref[cr_r, :, :] + stage_r_ref[h, :, :]
            rd_l.wait_recv()
            partial_ref[cr_l, :, :] = partial_ref[cr_l, :, :] + stage_l_ref[h, :, :]
            pending += [rd_r, rd_l]

        own_r = 2 * lax.rem(my + 1, N_DEV)
        own_l = 2 * lax.rem(my + N_DEV - 1, N_DEV) + 1
        out_ref[pl.ds(own_r * SUB, SUB), :] = partial_ref[own_r, :, :].astype(
            jnp.float32
        )
        out_ref[pl.ds(own_l * SUB, SUB), :] = partial_ref[own_l, :, :].astype(
            jnp.float32
        )

        for g in range(N_DEV - 1):
            cs_r = 2 * lax.rem(my + 1 + N_DEV - g, N_DEV)
            cr_r = 2 * lax.rem(my + N_DEV - g, N_DEV)
            cs_l = 2 * lax.rem(my + N_DEV - 1 + g, N_DEV) + 1
            cr_l = 2 * lax.rem(my + g, N_DEV) + 1
            ag_r = pltpu.make_async_remote_copy(
                src_ref=partial_ref.at[cs_r],
                dst_ref=partial_ref.at[cs_r],
                send_sem=ag_r_send.at[g],
                recv_sem=ag_r_recv.at[g],
                device_id=(right,),
                device_id_type=pl.DeviceIdType.MESH,
            )
            ag_l = pltpu.make_async_remote_copy(
                src_ref=partial_ref.at[cs_l],
                dst_ref=partial_ref.at[cs_l],
                send_sem=ag_l_send.at[g],
                recv_sem=ag_l_recv.at[g],
                device_id=(left,),
                device_id_type=pl.DeviceIdType.MESH,
            )
            ag_r.start()
            ag_l.start()
            ag_r.wait_recv()
            ag_l.wait_recv()
            out_ref[pl.ds(cr_r * SUB, SUB), :] = partial_ref[cr_r, :, :].astype(
                jnp.float32
            )
            out_ref[pl.ds(cr_l * SUB, SUB), :] = partial_ref[cr_l, :, :].astype(
                jnp.float32
            )
            pending += [ag_r, ag_l]

        for d in pending:
            d.wait_send()

        @functools.partial(
            pl.run_scoped, second_barrier=pltpu.SemaphoreType.REGULAR
        )
        def _(second_barrier):
            for nbr in (left, right):
                pl.semaphore_signal(
                    second_barrier,
                    inc=1,
                    device_id=(nbr,),
                    device_id_type=pl.DeviceIdType.MESH,
                )
            pl.semaphore_wait(second_barrier, 2)

    return pl.pallas_call(
        body,
        out_shape=jax.ShapeDtypeStruct((N_TOK, D_OUT), jnp.float32),
        in_specs=[pl.BlockSpec(memory_space=pltpu.VMEM)] * 4,
        out_specs=pl.BlockSpec(memory_space=pltpu.VMEM),
        scratch_shapes=[
            pltpu.VMEM((N_SUB, SUB, D_OUT), jnp.bfloat16),
            pltpu.VMEM((N_DEV - 1, SUB, D_OUT), jnp.bfloat16),
            pltpu.VMEM((N_DEV - 1, SUB, D_OUT), jnp.bfloat16),
        ]
        + [pltpu.SemaphoreType.DMA((N_DEV - 1,))] * 8,
        compiler_params=pltpu.CompilerParams(collective_id=0),
    )(x, router_W, route_idx, expert_W)


# device time: 87434 ns/iter; 1.4784x vs baseline; 1.0956x over previous
import functools

import jax
import jax.numpy as jnp
from jax import lax
from jax.experimental import pallas as pl
from jax.experimental.pallas import tpu as pltpu

N_DEV = 8
N_TOK = 2048
D_IN = 512
D_OUT = 1024
E_LOCAL = 4
N_SUB = 2 * N_DEV
SUB = N_TOK // N_SUB


def kernel(x, router_W, route_idx, expert_W):
    def body(
        x_ref,
        router_ref,
        route_ref,
        ew_ref,
        out_ref,
        partial_ref,
        stage_r_ref,
        stage_l_ref,
        rs_r_send,
        rs_r_recv,
        rs_l_send,
        rs_l_recv,
        ag_r_send,
        ag_r_recv,
        ag_l_send,
        ag_l_recv,
    ):
        my = lax.axis_index("i")
        left = lax.rem(my + N_DEV - 1, N_DEV)
        right = lax.rem(my + 1, N_DEV)

        bsem = pltpu.get_barrier_semaphore()
        for nbr in (left, right):
            pl.semaphore_signal(
                bsem, inc=1, device_id=(nbr,), device_id_type=pl.DeviceIdType.MESH
            )
        pl.semaphore_wait(bsem, 2)

        wbs = [ew_ref[e, :, :].astype(jnp.bfloat16) for e in range(E_LOCAL)]

        def compute_chunk(c):
            row0 = c * (2 * SUB)
            xb = x_ref[pl.ds(row0, 2 * SUB), :].astype(jnp.bfloat16)
            r = route_ref[pl.ds(row0, 2 * SUB), :]
            acc = jnp.zeros((2 * SUB, D_OUT), jnp.float32)
            for e in range(E_LOCAL):
                ge = my * E_LOCAL + e
                xm = jnp.where(r == ge, xb, jnp.zeros_like(xb))
                acc = acc + jnp.dot(xm, wbs[e], preferred_element_type=jnp.float32)
            accb = acc.astype(jnp.bfloat16)
            partial_ref[2 * c, :, :] = accb[:SUB, :]
            partial_ref[2 * c + 1, :, :] = accb[SUB:, :]

        compute_chunk(my)

        pending = []

        for h in range(N_DEV - 1):
            cs_r = 2 * lax.rem(my + N_DEV - h, N_DEV)
            cr_r = 2 * lax.rem(my + 2 * N_DEV - h - 1, N_DEV)
            cs_l = 2 * lax.rem(my + h, N_DEV) + 1
            cr_l = 2 * lax.rem(my + h + 1, N_DEV) + 1
            rd_r = pltpu.make_async_remote_copy(
                src_ref=partial_ref.at[cs_r],
                dst_ref=stage_r_ref.at[h],
                send_sem=rs_r_send.at[h],
                recv_sem=rs_r_recv.at[h],
                device_id=(right,),
                device_id_type=pl.DeviceIdType.MESH,
            )
            rd_l = pltpu.make_async_remote_copy(
                src_ref=partial_ref.at[cs_l],
                dst_ref=stage_l_ref.at[h],
                send_sem=rs_l_send.at[h],
                recv_sem=rs_l_recv.at[h],
                device_id=(left,),
                device_id_type=pl.DeviceIdType.MESH,
            )
            rd_r.start()
            rd_l.start()
            if h < 4:
                compute_chunk(lax.rem(my + 2 * N_DEV - h - 1, N_DEV))
                if h < 3:
                    compute_chunk(lax.rem(my + h + 1, N_DEV))
            rd_r.wait_recv()
            partial_ref[cr_r, :, :] = partial_ref[cr_r, :, :] + stage_r_ref[h, :, :]
            rd_l.wait_recv()
            partial_ref[cr_l, :, :] = partial_ref[cr_l, :, :] + stage_l_ref[h, :, :]
            pending += [rd_r, rd_l]

        def make_ag(g):
            cs_r = 2 * lax.rem(my + 1 + N_DEV - g, N_DEV)
            cs_l = 2 * lax.rem(my + N_DEV - 1 + g, N_DEV) + 1
            ag_r = pltpu.make_async_remote_copy(
                src_ref=partial_ref.at[cs_r],
                dst_ref=partial_ref.at[cs_r],
                send_sem=ag_r_send.at[g],
                recv_sem=ag_r_recv.at[g],
                device_id=(right,),
                device_id_type=pl.DeviceIdType.MESH,
            )
            ag_l = pltpu.make_async_remote_copy(
                src_ref=partial_ref.at[cs_l],
                dst_ref=partial_ref.at[cs_l],
                send_sem=ag_l_send.at[g],
                recv_sem=ag_l_recv.at[g],
                device_id=(left,),
                device_id_type=pl.DeviceIdType.MESH,
            )
            ag_r.start()
            ag_l.start()
            pending.append(ag_r)
            pending.append(ag_l)
            return ag_r, ag_l

        ag_cur = make_ag(0)

        own_r = 2 * lax.rem(my + 1, N_DEV)
        own_l = 2 * lax.rem(my + N_DEV - 1, N_DEV) + 1
        out_ref[pl.ds(own_r * SUB, SUB), :] = partial_ref[own_r, :, :].astype(
            jnp.float32
        )
        out_ref[pl.ds(own_l * SUB, SUB), :] = partial_ref[own_l, :, :].astype(
            jnp.float32
        )

        for g in range(N_DEV - 1):
            cr_r = 2 * lax.rem(my + N_DEV - g, N_DEV)
            cr_l = 2 * lax.rem(my + g, N_DEV) + 1
            ag_r, ag_l = ag_cur
            ag_r.wait_recv()
            ag_l.wait_recv()
            if g < N_DEV - 2:
                ag_cur = make_ag(g + 1)
            out_ref[pl.ds(cr_r * SUB, SUB), :] = partial_ref[cr_r, :, :].astype(
                jnp.float32
            )
            out_ref[pl.ds(cr_l * SUB, SUB), :] = partial_ref[cr_l, :, :].astype(
                jnp.float32
            )

        for d in pending:
            d.wait_send()

        @functools.partial(
            pl.run_scoped, second_barrier=pltpu.SemaphoreType.REGULAR
        )
        def _(second_barrier):
            for nbr in (left, right):
                pl.semaphore_signal(
                    second_barrier,
                    inc=1,
                    device_id=(nbr,),
                    device_id_type=pl.DeviceIdType.MESH,
                )
            pl.semaphore_wait(second_barrier, 2)

    return pl.pallas_call(
        body,
        out_shape=jax.ShapeDtypeStruct((N_TOK, D_OUT), jnp.float32),
        in_specs=[pl.BlockSpec(memory_space=pltpu.VMEM)] * 4,
        out_specs=pl.BlockSpec(memory_space=pltpu.VMEM),
        scratch_shapes=[
            pltpu.VMEM((N_SUB, SUB, D_OUT), jnp.bfloat16),
            pltpu.VMEM((N_DEV - 1, SUB, D_OUT), jnp.bfloat16),
            pltpu.VMEM((N_DEV - 1, SUB, D_OUT), jnp.bfloat16),
        ]
        + [pltpu.SemaphoreType.DMA((N_DEV - 1,))] * 8,
        compiler_params=pltpu.CompilerParams(collective_id=0),
    )(x, router_W, route_idx, expert_W)


# device time: 70319 ns/iter; 1.8382x vs baseline; 1.2434x over previous
import functools

import jax
import jax.numpy as jnp
from jax import lax
from jax.experimental import pallas as pl
from jax.experimental.pallas import tpu as pltpu

N_DEV = 8
N_TOK = 2048
D_IN = 512
D_OUT = 1024
E_LOCAL = 4
N_SUB = 2 * N_DEV
SUB = N_TOK // N_SUB
HALF = N_TOK // 2


def kernel(x, router_W, route_idx, expert_W):
    def body(
        x_ref,
        router_ref,
        route_ref,
        ew_ref,
        out_ref,
        partial_ref,
        stg_a_ref,
        stg_b_ref,
        rs_a_send,
        rs_a_recv,
        rs_b_send,
        rs_b_recv,
        ag_a_send,
        ag_a_recv,
        ag_b_send,
        ag_b_recv,
    ):
        my = lax.axis_index("i")
        b0 = lax.rem(my, 2)
        b1 = lax.rem(lax.div(my, 2), 2)
        b2 = lax.div(my, 4)

        xor1 = my + 1 - 2 * b0
        m4 = my - 4 * b2
        xor3 = my + 3 - 2 * m4
        xor4 = my + 4 - 8 * b2

        g0 = b0 + b1 - 2 * b0 * b1
        ga = (g0, b1, b2)
        gb = (b2, g0, b1)
        la = 4 * ga[0] + 2 * ga[1] + ga[2]
        lb = 4 * gb[0] + 2 * gb[1] + gb[2]

        def lbl(q, g):
            v = 0
            for w, gk in zip((4, 2, 1), g):
                v = v + (w * (1 - gk) if q & w else w * gk)
            return v

        bsem = pltpu.get_barrier_semaphore()
        for nbr in (xor1, xor3, xor4):
            pl.semaphore_signal(
                bsem, inc=1, device_id=(nbr,), device_id_type=pl.DeviceIdType.MESH
            )
        pl.semaphore_wait(bsem, 3)

        wbs = [ew_ref[e, :, :].astype(jnp.bfloat16) for e in range(E_LOCAL)]

        def compute_block(c, slot0, slot1):
            row0 = c * (2 * SUB)
            xb = x_ref[pl.ds(row0, 2 * SUB), :].astype(jnp.bfloat16)
            r = route_ref[pl.ds(row0, 2 * SUB), :]
            acc = jnp.zeros((2 * SUB, D_OUT), jnp.float32)
            for e in range(E_LOCAL):
                ge = my * E_LOCAL + e
                xm = jnp.where(r == ge, xb, jnp.zeros_like(xb))
                acc = acc + jnp.dot(xm, wbs[e], preferred_element_type=jnp.float32)
            accb = acc.astype(jnp.bfloat16)
            partial_ref[slot0, :, :] = accb[:SUB, :]
            partial_ref[slot1, :, :] = accb[SUB:, :]

        def sl(base, j, k, g):
            return (
                base
                + 2 * (g[1] if j == 0 else 1 - g[1])
                + (g[2] if k == 0 else 1 - g[2])
            )

        pending = []

        def start(src, dst, ssem, rsem, partner):
            d = pltpu.make_async_remote_copy(
                src_ref=src,
                dst_ref=dst,
                send_sem=ssem,
                recv_sem=rsem,
                device_id=(partner,),
                device_id_type=pl.DeviceIdType.MESH,
            )
            d.start()
            pending.append(d)
            return d

        for j in range(2):
            compute_block(
                2 * (1 - ga[0]) + j, sl(4, j, 0, ga), sl(4, j, 1, ga)
            )
        rs_a0 = start(
            partial_ref.at[pl.ds(4, 4)], stg_a_ref.at[pl.ds(0, 4)],
            rs_a_send.at[0], rs_a_recv.at[0], xor1,
        )
        for j in range(2):
            compute_block(
                4 + 2 * (1 - gb[0]) + j, sl(12, j, 0, gb), sl(12, j, 1, gb)
            )
        rs_b0 = start(
            partial_ref.at[pl.ds(12, 4)], stg_b_ref.at[pl.ds(0, 4)],
            rs_b_send.at[0], rs_b_recv.at[0], xor4,
        )
        for j in range(2):
            compute_block(2 * ga[0] + j, sl(0, j, 0, ga), sl(0, j, 1, ga))
        for j in range(2):
            compute_block(
                4 + 2 * gb[0] + j, sl(8, j, 0, gb), sl(8, j, 1, gb)
            )

        rs_a0.wait_recv()
        partial_ref[pl.ds(0, 4), :, :] = (
            partial_ref[pl.ds(0, 4), :, :] + stg_a_ref[pl.ds(0, 4), :, :]
        )
        rs_b0.wait_recv()
        partial_ref[pl.ds(8, 4), :, :] = (
            partial_ref[pl.ds(8, 4), :, :] + stg_b_ref[pl.ds(0, 4), :, :]
        )

        rs_a1 = start(
            partial_ref.at[pl.ds(2, 2)], stg_a_ref.at[pl.ds(4, 2)],
            rs_a_send.at[1], rs_a_recv.at[1], xor3,
        )
        rs_b1 = start(
            partial_ref.at[pl.ds(10, 2)], stg_b_ref.at[pl.ds(4, 2)],
            rs_b_send.at[1], rs_b_recv.at[1], xor1,
        )
        rs_a1.wait_recv()
        partial_ref[pl.ds(0, 2), :, :] = (
            partial_ref[pl.ds(0, 2), :, :] + stg_a_ref[pl.ds(4, 2), :, :]
        )
        rs_b1.wait_recv()
        partial_ref[pl.ds(8, 2), :, :] = (
            partial_ref[pl.ds(8, 2), :, :] + stg_b_ref[pl.ds(4, 2), :, :]
        )

        rs_a2 = start(
            partial_ref.at[pl.ds(1, 1)], stg_a_ref.at[pl.ds(6, 1)],
            rs_a_send.at[2], rs_a_recv.at[2], xor4,
        )
        rs_b2 = start(
            partial_ref.at[pl.ds(9, 1)], stg_b_ref.at[pl.ds(6, 1)],
            rs_b_send.at[2], rs_b_recv.at[2], xor3,
        )
        rs_a2.wait_recv()
        partial_ref[pl.ds(0, 1), :, :] = (
            partial_ref[pl.ds(0, 1), :, :] + stg_a_ref[pl.ds(6, 1), :, :]
        )
        rs_b2.wait_recv()
        partial_ref[pl.ds(8, 1), :, :] = (
            partial_ref[pl.ds(8, 1), :, :] + stg_b_ref[pl.ds(6, 1), :, :]
        )

        ag_a0 = start(
            partial_ref.at[pl.ds(0, 1)], partial_ref.at[pl.ds(1, 1)],
            ag_a_send.at[0], ag_a_recv.at[0], xor4,
        )
        ag_b0 = start(
            partial_ref.at[pl.ds(8, 1)], partial_ref.at[pl.ds(9, 1)],
            ag_b_send.at[0], ag_b_recv.at[0], xor3,
        )
        out_ref[pl.ds(la * SUB, SUB), :] = partial_ref[0, :, :].astype(jnp.float32)
        out_ref[pl.ds(HALF + lb * SUB, SUB), :] = partial_ref[8, :, :].astype(
            jnp.float32
        )

        ag_a0.wait_recv()
        ag_b0.wait_recv()
        ag_a1 = start(
            partial_ref.at[pl.ds(0, 2)], partial_ref.at[pl.ds(2, 2)],
            ag_a_send.at[1], ag_a_recv.at[1], xor3,
        )
        ag_b1 = start(
            partial_ref.at[pl.ds(8, 2)], partial_ref.at[pl.ds(10, 2)],
            ag_b_send.at[1], ag_b_recv.at[1], xor1,
        )
        out_ref[pl.ds(lbl(1, ga) * SUB, SUB), :] = partial_ref[1, :, :].astype(
            jnp.float32
        )
        out_ref[pl.ds(HALF + lbl(1, gb) * SUB, SUB), :] = partial_ref[
            9, :, :
        ].astype(jnp.float32)

        ag_a1.wait_recv()
        ag_b1.wait_recv()
        ag_a2 = start(
            partial_ref.at[pl.ds(0, 4)], partial_ref.at[pl.ds(4, 4)],
            ag_a_send.at[2], ag_a_recv.at[2], xor1,
        )
        ag_b2 = start(
            partial_ref.at[pl.ds(8, 4)], partial_ref.at[pl.ds(12, 4)],
            ag_b_send.at[2], ag_b_recv.at[2], xor4,
        )
        for q in (2, 3):
            out_ref[pl.ds(lbl(q, ga) * SUB, SUB), :] = partial_ref[
                q, :, :
            ].astype(jnp.float32)
            out_ref[pl.ds(HALF + lbl(q, gb) * SUB, SUB), :] = partial_ref[
                8 + q, :, :
            ].astype(jnp.float32)

        ag_a2.wait_recv()
        ag_b2.wait_recv()
        for q in (4, 5, 6, 7):
            out_ref[pl.ds(lbl(q, ga) * SUB, SUB), :] = partial_ref[
                q, :, :
            ].astype(jnp.float32)
            out_ref[pl.ds(HALF + lbl(q, gb) * SUB, SUB), :] = partial_ref[
                8 + q, :, :
            ].astype(jnp.float32)

        for d in pending:
            d.wait_send()

        @functools.partial(
            pl.run_scoped, second_barrier=pltpu.SemaphoreType.REGULAR
        )
        def _(second_barrier):
            for nbr in (xor1, xor3, xor4):
                pl.semaphore_signal(
                    second_barrier,
                    inc=1,
                    device_id=(nbr,),
                    device_id_type=pl.DeviceIdType.MESH,
                )
            pl.semaphore_wait(second_barrier, 3)

    return pl.pallas_call(
        body,
        out_shape=jax.ShapeDtypeStruct((N_TOK, D_OUT), jnp.float32),
        in_specs=[pl.BlockSpec(memory_space=pltpu.VMEM)] * 4,
        out_specs=pl.BlockSpec(memory_space=pltpu.VMEM),
        scratch_shapes=[
            pltpu.VMEM((N_SUB, SUB, D_OUT), jnp.bfloat16),
            pltpu.VMEM((N_DEV - 1, SUB, D_OUT), jnp.bfloat16),
            pltpu.VMEM((N_DEV - 1, SUB, D_OUT), jnp.bfloat16),
        ]
        + [pltpu.SemaphoreType.DMA((3,))] * 8,
        compiler_params=pltpu.CompilerParams(collective_id=0),
    )(x, router_W, route_idx, expert_W)


# device time: 59342 ns/iter; 2.1782x vs baseline; 1.1850x over previous
import functools

import jax
import jax.numpy as jnp
from jax import lax
from jax.experimental import pallas as pl
from jax.experimental.pallas import tpu as pltpu

N_DEV = 8
N_TOK = 2048
D_IN = 512
D_OUT = 1024
E_LOCAL = 4

PARTS = (
    (0, 96, (1, 3, 4)),
    (768, 80, (3, 4, 1)),
    (1408, 80, (4, 1, 3)),
)
STG_OFF = (0, 4, 6)
RS_CNT = (4, 2, 1)
AG_CNT = (1, 2, 4)


def kernel(x, router_W, route_idx, expert_W):
    def body(x_ref, router_ref, route_ref, ew_ref, out_ref, partial_ref, *scr):
        stg = {0: scr[0], 1: scr[1], 2: scr[2]}
        sems = scr[3:]
        rs_send = {i: sems[4 * i] for i in range(3)}
        rs_recv = {i: sems[4 * i + 1] for i in range(3)}
        ag_send = {i: sems[4 * i + 2] for i in range(3)}
        ag_recv = {i: sems[4 * i + 3] for i in range(3)}

        my = lax.axis_index("i")
        b0 = lax.rem(my, 2)
        b1 = lax.rem(lax.div(my, 2), 2)
        b2 = lax.div(my, 4)
        g0 = b0 + b1 - 2 * b0 * b1

        m4 = my - 4 * b2
        partner = {
            1: my + 1 - 2 * b0,
            3: my + 3 - 2 * m4,
            4: my + 4 - 8 * b2,
        }
        keys = {0: (g0, b1, b2), 1: (b1, b2, g0), 2: (b2, g0, b1)}

        def lbl(q, g):
            v = 0
            for w, gk in zip((4, 2, 1), g):
                v = v + (w * (1 - gk) if q & w else w * gk)
            return v

        bsem = pltpu.get_barrier_semaphore()
        for mask in (1, 3, 4):
            pl.semaphore_signal(
                bsem,
                inc=1,
                device_id=(partner[mask],),
                device_id_type=pl.DeviceIdType.MESH,
            )
        pl.semaphore_wait(bsem, 3)

        wbs = [ew_ref[e, :, :].astype(jnp.bfloat16) for e in range(E_LOCAL)]

        def compute_span(pi, v0):
            base, sub, _ = PARTS[pi]
            g = keys[pi]
            row0 = base + v0 * (4 * sub)
            xb = x_ref[pl.ds(row0, 4 * sub), :].astype(jnp.bfloat16)
            r = route_ref[pl.ds(row0, 4 * sub), :]
            acc = jnp.zeros((4 * sub, D_OUT), jnp.float32)
            for e in range(E_LOCAL):
                ge = my * E_LOCAL + e
                xm = jnp.where(r == ge, xb, jnp.zeros_like(xb))
                acc = acc + jnp.dot(xm, wbs[e], preferred_element_type=jnp.float32)
            accb = acc.astype(jnp.bfloat16)
            s0 = v0 + g[0] - 2 * v0 * g[0]
            for j in (0, 1):
                for m in (0, 1):
                    slot = (
                        4 * s0
                        + 2 * (g[1] if j == 0 else 1 - g[1])
                        + (g[2] if m == 0 else 1 - g[2])
                    )
                    partial_ref[pl.ds(base + slot * sub, sub), :] = accb[
                        (2 * j + m) * sub : (2 * j + m + 1) * sub, :
                    ]

        pending = []

        def start(src, dst, ssem, rsem, mask):
            d = pltpu.make_async_remote_copy(
                src_ref=src,
                dst_ref=dst,
                send_sem=ssem,
                recv_sem=rsem,
                device_id=(partner[mask],),
                device_id_type=pl.DeviceIdType.MESH,
            )
            d.start()
            pending.append(d)
            return d

        def rs_start(pi, k):
            base, sub, masks = PARTS[pi]
            cnt = RS_CNT[k]
            return start(
                partial_ref.at[pl.ds(base + cnt * sub, cnt * sub)],
                stg[pi].at[pl.ds(STG_OFF[k] * sub, cnt * sub)],
                rs_send[pi].at[k],
                rs_recv[pi].at[k],
                masks[k],
            )

        def rs_add(pi, k):
            base, sub, _ = PARTS[pi]
            cnt = RS_CNT[k]
            partial_ref[pl.ds(base, cnt * sub), :] = (
                partial_ref[pl.ds(base, cnt * sub), :]
                + stg[pi][pl.ds(STG_OFF[k] * sub, cnt * sub), :]
            )

        def ag_start(pi, k):
            base, sub, masks = PARTS[pi]
            cnt = AG_CNT[k]
            return start(
                partial_ref.at[pl.ds(base, cnt * sub)],
                partial_ref.at[pl.ds(base + cnt * sub, cnt * sub)],
                ag_send[pi].at[k],
                ag_recv[pi].at[k],
                masks[2 - k],
            )

        def store(pi, q):
            base, sub, _ = PARTS[pi]
            out_ref[pl.ds(base + lbl(q, keys[pi]) * sub, sub), :] = partial_ref[
                pl.ds(base + q * sub, sub), :
            ].astype(jnp.float32)

        rs0 = {}
        for pi in range(3):
            compute_span(pi, 1 - keys[pi][0])
            rs0[pi] = rs_start(pi, 0)
        for pi in range(3):
            compute_span(pi, keys[pi][0])
        for pi in range(3):
            rs0[pi].wait_recv()
            rs_add(pi, 0)

        for k in (1, 2):
            d = {pi: rs_start(pi, k) for pi in range(3)}
            for pi in range(3):
                d[pi].wait_recv()
                rs_add(pi, k)

        ag_cur = {pi: ag_start(pi, 0) for pi in range(3)}
        for pi in range(3):
            store(pi, 0)
        for k in (0, 1, 2):
            for pi in range(3):
                ag_cur[pi].wait_recv()
            if k < 2:
                ag_nxt = {pi: ag_start(pi, k + 1) for pi in range(3)}
            for pi in range(3):
                for q in range(AG_CNT[k], 2 * AG_CNT[k]):
                    store(pi, q)
            if k < 2:
                ag_cur = ag_nxt

        for d in pending:
            d.wait_send()

        @functools.partial(
            pl.run_scoped, second_barrier=pltpu.SemaphoreType.REGULAR
        )
        def _(second_barrier):
            for mask in (1, 3, 4):
                pl.semaphore_signal(
                    second_barrier,
                    inc=1,
                    device_id=(partner[mask],),
                    device_id_type=pl.DeviceIdType.MESH,
                )
            pl.semaphore_wait(second_barrier, 3)

    sem_scratch = []
    for _ in range(3):
        sem_scratch += [pltpu.SemaphoreType.DMA((3,))] * 4

    return pl.pallas_call(
        body,
        out_shape=jax.ShapeDtypeStruct((N_TOK, D_OUT), jnp.float32),
        in_specs=[pl.BlockSpec(memory_space=pltpu.VMEM)] * 4,
        out_specs=pl.BlockSpec(memory_space=pltpu.VMEM),
        scratch_shapes=[
            pltpu.VMEM((N_TOK, D_OUT), jnp.bfloat16),
            pltpu.VMEM((7 * 96, D_OUT), jnp.bfloat16),
            pltpu.VMEM((7 * 80, D_OUT), jnp.bfloat16),
            pltpu.VMEM((7 * 80, D_OUT), jnp.bfloat16),
        ]
        + sem_scratch,
        compiler_params=pltpu.CompilerParams(collective_id=0),
    )(x, router_W, route_idx, expert_W)


# device time: 57477 ns/iter; 2.2489x vs baseline; 1.0324x over previous
import functools

import jax
import jax.numpy as jnp
from jax import lax
from jax.experimental import pallas as pl
from jax.experimental.pallas import tpu as pltpu

N_DEV = 8
N_TOK = 2048
D_IN = 512
D_OUT = 1024
E_LOCAL = 4

PARTS = (
    (0, 96, (1, 3, 4)),
    (768, 80, (3, 4, 1)),
    (1408, 80, (4, 1, 3)),
)
STG_OFF = (0, 4, 6)
RS_CNT = (4, 2, 1)
AG_CNT = (1, 2, 4)


def kernel(x, router_W, route_idx, expert_W):
    def body(x_ref, router_ref, route_ref, ew_ref, out_ref, partial_ref, *scr):
        stg = {0: scr[0], 1: scr[1], 2: scr[2]}
        sems = scr[3:]
        rs_send = {i: sems[4 * i] for i in range(3)}
        rs_recv = {i: sems[4 * i + 1] for i in range(3)}
        ag_send = {i: sems[4 * i + 2] for i in range(3)}
        ag_recv = {i: sems[4 * i + 3] for i in range(3)}

        my = lax.axis_index("i")
        b0 = lax.rem(my, 2)
        b1 = lax.rem(lax.div(my, 2), 2)
        b2 = lax.div(my, 4)
        g0 = b0 + b1 - 2 * b0 * b1

        m4 = my - 4 * b2
        partner = {
            1: my + 1 - 2 * b0,
            3: my + 3 - 2 * m4,
            4: my + 4 - 8 * b2,
        }
        keys = {0: (g0, b1, b2), 1: (b1, b2, g0), 2: (b2, g0, b1)}

        def lbl(q, g):
            v = 0
            for w, gk in zip((4, 2, 1), g):
                v = v + (w * (1 - gk) if q & w else w * gk)
            return v

        bsem = pltpu.get_barrier_semaphore()
        for mask in (1, 3, 4):
            pl.semaphore_signal(
                bsem,
                inc=1,
                device_id=(partner[mask],),
                device_id_type=pl.DeviceIdType.MESH,
            )
        pl.semaphore_wait(bsem, 3)

        wbs = [ew_ref[e, :, :].astype(jnp.bfloat16) for e in range(E_LOCAL)]

        def compute_span(pi, v0):
            base, sub, _ = PARTS[pi]
            g = keys[pi]
            row0 = base + v0 * (4 * sub)
            xb = x_ref[pl.ds(row0, 4 * sub), :].astype(jnp.bfloat16)
            r = route_ref[pl.ds(row0, 4 * sub), :]
            acc = jnp.zeros((4 * sub, D_OUT), jnp.float32)
            for e in range(E_LOCAL):
                ge = my * E_LOCAL + e
                xm = jnp.where(r == ge, xb, jnp.zeros_like(xb))
                acc = acc + jnp.dot(xm, wbs[e], preferred_element_type=jnp.float32)
            accb = acc.astype(jnp.bfloat16)
            s0 = v0 + g[0] - 2 * v0 * g[0]
            for j in (0, 1):
                for m in (0, 1):
                    slot = (
                        4 * s0
                        + 2 * (g[1] if j == 0 else 1 - g[1])
                        + (g[2] if m == 0 else 1 - g[2])
                    )
                    partial_ref[pl.ds(base + slot * sub, sub), :] = accb[
                        (2 * j + m) * sub : (2 * j + m + 1) * sub, :
                    ]

        pending = []

        def start(src, dst, ssem, rsem, mask):
            d = pltpu.make_async_remote_copy(
                src_ref=src,
                dst_ref=dst,
                send_sem=ssem,
                recv_sem=rsem,
                device_id=(partner[mask],),
                device_id_type=pl.DeviceIdType.MESH,
            )
            d.start()
            pending.append(d)
            return d

        def rs_start(pi, k):
            base, sub, masks = PARTS[pi]
            cnt = RS_CNT[k]
            return start(
                partial_ref.at[pl.ds(base + cnt * sub, cnt * sub)],
                stg[pi].at[pl.ds(STG_OFF[k] * sub, cnt * sub)],
                rs_send[pi].at[k],
                rs_recv[pi].at[k],
                masks[k],
            )

        def rs_add(pi, k):
            base, sub, _ = PARTS[pi]
            cnt = RS_CNT[k]
            partial_ref[pl.ds(base, cnt * sub), :] = (
                partial_ref[pl.ds(base, cnt * sub), :]
                + stg[pi][pl.ds(STG_OFF[k] * sub, cnt * sub), :]
            )

        def ag_start(pi, k):
            base, sub, masks = PARTS[pi]
            cnt = AG_CNT[k]
            g = keys[pi]
            held = 4 * g[0] + (2 * g[1] if cnt < 4 else 0) + (g[2] if cnt < 2 else 0)
            blk = out_ref.at[pl.ds(base + held * sub, cnt * sub)]
            return start(
                blk, blk, ag_send[pi].at[k], ag_recv[pi].at[k], masks[2 - k]
            )

        rs0 = {}
        for pi in range(3):
            compute_span(pi, 1 - keys[pi][0])
            rs0[pi] = rs_start(pi, 0)
        for pi in range(3):
            compute_span(pi, keys[pi][0])
        for pi in range(3):
            rs0[pi].wait_recv()
            rs_add(pi, 0)

        for k in (1, 2):
            d = {pi: rs_start(pi, k) for pi in range(3)}
            for pi in range(3):
                d[pi].wait_recv()
                rs_add(pi, k)

        for pi in range(3):
            base, sub, _ = PARTS[pi]
            g = keys[pi]
            held = 4 * g[0] + 2 * g[1] + g[2]
            out_ref[pl.ds(base + held * sub, sub), :] = partial_ref[
                pl.ds(base, sub), :
            ]
        for k in (0, 1, 2):
            d = {pi: ag_start(pi, k) for pi in range(3)}
            for pi in range(3):
                d[pi].wait_recv()

        for d in pending:
            d.wait_send()

        @functools.partial(
            pl.run_scoped, second_barrier=pltpu.SemaphoreType.REGULAR
        )
        def _(second_barrier):
            for mask in (1, 3, 4):
                pl.semaphore_signal(
                    second_barrier,
                    inc=1,
                    device_id=(partner[mask],),
                    device_id_type=pl.DeviceIdType.MESH,
                )
            pl.semaphore_wait(second_barrier, 3)

    sem_scratch = []
    for _ in range(3):
        sem_scratch += [pltpu.SemaphoreType.DMA((3,))] * 4

    return pl.pallas_call(
        body,
        out_shape=jax.ShapeDtypeStruct((N_TOK, D_OUT), jnp.bfloat16),
        in_specs=[pl.BlockSpec(memory_space=pltpu.VMEM)] * 4,
        out_specs=pl.BlockSpec(memory_space=pltpu.VMEM),
        scratch_shapes=[
            pltpu.VMEM((N_TOK, D_OUT), jnp.bfloat16),
            pltpu.VMEM((7 * 96, D_OUT), jnp.bfloat16),
            pltpu.VMEM((7 * 80, D_OUT), jnp.bfloat16),
            pltpu.VMEM((7 * 80, D_OUT), jnp.bfloat16),
        ]
        + sem_scratch,
        compiler_params=pltpu.CompilerParams(collective_id=0),
    )(x, router_W, route_idx, expert_W)


# device time: 56663 ns/iter; 2.2812x vs baseline; 1.0144x over previous
import functools

import jax
import jax.numpy as jnp
from jax import lax
from jax.experimental import pallas as pl
from jax.experimental.pallas import tpu as pltpu

N_DEV = 8
N_TOK = 2048
D_IN = 512
D_OUT = 1024
E_LOCAL = 4

PARTS = (
    (0, 96, (1, 3, 4)),
    (768, 80, (3, 4, 1)),
    (1408, 80, (4, 1, 3)),
)
STG_OFF = (0, 4, 6)
RS_CNT = (4, 2, 1)
AG_CNT = (1, 2, 4)


def kernel(x, router_W, route_idx, expert_W):
    def body(x_ref, router_ref, route_ref, ew_ref, out_ref, partial_ref, *scr):
        stg = {0: scr[0], 1: scr[1], 2: scr[2]}
        sems = scr[3:]
        rs_send = {i: sems[4 * i] for i in range(3)}
        rs_recv = {i: sems[4 * i + 1] for i in range(3)}
        ag_send = {i: sems[4 * i + 2] for i in range(3)}
        ag_recv = {i: sems[4 * i + 3] for i in range(3)}

        my = lax.axis_index("i")
        b0 = lax.rem(my, 2)
        b1 = lax.rem(lax.div(my, 2), 2)
        b2 = lax.div(my, 4)
        g0 = b0 + b1 - 2 * b0 * b1

        m4 = my - 4 * b2
        partner = {
            1: my + 1 - 2 * b0,
            3: my + 3 - 2 * m4,
            4: my + 4 - 8 * b2,
        }
        keys = {0: (g0, b1, b2), 1: (b1, b2, g0), 2: (b2, g0, b1)}

        def lbl(q, g):
            v = 0
            for w, gk in zip((4, 2, 1), g):
                v = v + (w * (1 - gk) if q & w else w * gk)
            return v

        bsem = pltpu.get_barrier_semaphore()
        for mask in (1, 3, 4):
            pl.semaphore_signal(
                bsem,
                inc=1,
                device_id=(partner[mask],),
                device_id_type=pl.DeviceIdType.MESH,
            )
        pl.semaphore_wait(bsem, 3)

        wbs = [ew_ref[e, :, :].astype(jnp.bfloat16) for e in range(E_LOCAL)]

        def compute_span(pi, v0):
            base, sub, _ = PARTS[pi]
            g = keys[pi]
            row0 = base + v0 * (4 * sub)
            xb = x_ref[pl.ds(row0, 4 * sub), :].astype(jnp.bfloat16)
            r = route_ref[pl.ds(row0, 4 * sub), :]
            acc = jnp.zeros((4 * sub, D_OUT), jnp.float32)
            for e in range(E_LOCAL):
                ge = my * E_LOCAL + e
                xm = jnp.where(r == ge, xb, jnp.zeros_like(xb))
                acc = acc + jnp.dot(xm, wbs[e], preferred_element_type=jnp.float32)
            accb = acc.astype(jnp.bfloat16)
            s0 = v0 + g[0] - 2 * v0 * g[0]
            for j in (0, 1):
                for m in (0, 1):
                    slot = (
                        4 * s0
                        + 2 * (g[1] if j == 0 else 1 - g[1])
                        + (g[2] if m == 0 else 1 - g[2])
                    )
                    partial_ref[pl.ds(base + slot * sub, sub), :] = accb[
                        (2 * j + m) * sub : (2 * j + m + 1) * sub, :
                    ]

        pending = []

        def start(src, dst, ssem, rsem, mask):
            d = pltpu.make_async_remote_copy(
                src_ref=src,
                dst_ref=dst,
                send_sem=ssem,
                recv_sem=rsem,
                device_id=(partner[mask],),
                device_id_type=pl.DeviceIdType.MESH,
            )
            d.start()
            pending.append(d)
            return d

        def rs_start(pi, k):
            base, sub, masks = PARTS[pi]
            cnt = RS_CNT[k]
            return start(
                partial_ref.at[pl.ds(base + cnt * sub, cnt * sub)],
                stg[pi].at[pl.ds(STG_OFF[k] * sub, cnt * sub)],
                rs_send[pi].at[k],
                rs_recv[pi].at[k],
                masks[k],
            )

        def rs_add(pi, k):
            base, sub, _ = PARTS[pi]
            cnt = RS_CNT[k]
            partial_ref[pl.ds(base, cnt * sub), :] = (
                partial_ref[pl.ds(base, cnt * sub), :]
                + stg[pi][pl.ds(STG_OFF[k] * sub, cnt * sub), :]
            )

        def ag_start(pi, k):
            base, sub, masks = PARTS[pi]
            cnt = AG_CNT[k]
            g = keys[pi]
            held = 4 * g[0] + (2 * g[1] if cnt < 4 else 0) + (g[2] if cnt < 2 else 0)
            blk = out_ref.at[pl.ds(base + held * sub, cnt * sub)]
            return start(
                blk, blk, ag_send[pi].at[k], ag_recv[pi].at[k], masks[2 - k]
            )

        rs0 = {}
        for pi in range(3):
            compute_span(pi, 1 - keys[pi][0])
            rs0[pi] = rs_start(pi, 0)
        for pi in range(3):
            compute_span(pi, keys[pi][0])
        for pi in range(3):
            rs0[pi].wait_recv()
            rs_add(pi, 0)

        d = {pi: rs_start(pi, 1) for pi in range(3)}
        for pi in range(3):
            d[pi].wait_recv()
            rs_add(pi, 1)

        ex = {}
        for pi in range(3):
            base, sub, masks = PARTS[pi]
            ex[pi] = (
                start(
                    partial_ref.at[pl.ds(base + sub, sub)],
                    stg[pi].at[pl.ds(6 * sub, sub)],
                    rs_send[pi].at[2],
                    rs_recv[pi].at[2],
                    masks[2],
                ),
                start(
                    partial_ref.at[pl.ds(base, sub)],
                    stg[pi].at[pl.ds(7 * sub, sub)],
                    rs_send[pi].at[3],
                    rs_recv[pi].at[3],
                    masks[2],
                ),
            )
        for pi in range(3):
            base, sub, _ = PARTS[pi]
            g = keys[pi]
            ex[pi][0].wait_recv()
            ex[pi][1].wait_recv()
            pair = 4 * g[0] + 2 * g[1]
            for q in (0, 1):
                v2 = g[2] if q == 0 else 1 - g[2]
                out_ref[pl.ds(base + (pair + v2) * sub, sub), :] = (
                    partial_ref[pl.ds(base + q * sub, sub), :]
                    + stg[pi][pl.ds((6 + q) * sub, sub), :]
                )

        for k in (1, 2):
            d = {pi: ag_start(pi, k) for pi in range(3)}
            for pi in range(3):
                d[pi].wait_recv()

        for d in pending:
            d.wait_send()

        @functools.partial(
            pl.run_scoped, second_barrier=pltpu.SemaphoreType.REGULAR
        )
        def _(second_barrier):
            for mask in (1, 3, 4):
                pl.semaphore_signal(
                    second_barrier,
                    inc=1,
                    device_id=(partner[mask],),
                    device_id_type=pl.DeviceIdType.MESH,
                )
            pl.semaphore_wait(second_barrier, 3)

    sem_scratch = []
    for _ in range(3):
        sem_scratch += [
            pltpu.SemaphoreType.DMA((4,)),
            pltpu.SemaphoreType.DMA((4,)),
            pltpu.SemaphoreType.DMA((3,)),
            pltpu.SemaphoreType.DMA((3,)),
        ]

    return pl.pallas_call(
        body,
        out_shape=jax.ShapeDtypeStruct((N_TOK, D_OUT), jnp.bfloat16),
        in_specs=[pl.BlockSpec(memory_space=pltpu.VMEM)] * 4,
        out_specs=pl.BlockSpec(memory_space=pltpu.VMEM),
        scratch_shapes=[
            pltpu.VMEM((N_TOK, D_OUT), jnp.bfloat16),
            pltpu.VMEM((8 * 96, D_OUT), jnp.bfloat16),
            pltpu.VMEM((8 * 80, D_OUT), jnp.bfloat16),
            pltpu.VMEM((8 * 80, D_OUT), jnp.bfloat16),
        ]
        + sem_scratch,
        compiler_params=pltpu.CompilerParams(collective_id=0),
    )(x, router_W, route_idx, expert_W)
